# baseline (speedup 1.0000x reference)
import functools

import jax
import jax.numpy as jnp
from jax import lax
from jax.experimental import pallas as pl
from jax.experimental.pallas import tpu as pltpu
from jax.experimental.pallas import tpu_sc as plsc

_LANES = 16
_SUB = 128


def _build(B, H, W, C, P):
    NC, NS = 2, 16
    NW = NC * NS
    n_pts = B * P
    ppw = n_pts // NW
    nsub = ppw // _SUB
    nvec = ppw // _LANES
    cvec = C // _LANES
    assert n_pts % NW == 0 and ppw % _SUB == 0 and C % _LANES == 0
    assert P % ppw == 0

    mesh = plsc.VectorSubcoreMesh(
        core_axis_name="c", subcore_axis_name="s", num_cores=NC, num_subcores=NS)

    @functools.partial(
        pl.kernel,
        out_type=jax.ShapeDtypeStruct((n_pts, C), jnp.float32),
        mesh=mesh,
        compiler_params=pltpu.CompilerParams(
            needs_layout_passes=False, use_tc_tiling_on_sc=False),
        scratch_types=[
            pltpu.VMEM((2 * ppw,), jnp.float32),
            pltpu.VMEM((4, nsub, _SUB), jnp.int32),
            pltpu.VMEM((4 * ppw,), jnp.float32),
            pltpu.VMEM((4, _SUB, C), jnp.float32),
            pltpu.VMEM((_SUB, C), jnp.float32),
            pltpu.SemaphoreType.DMA,
        ],
    )
    def point_sample(feat_hbm, grid_hbm, out_hbm, gxy, idxb, wb, rows, outb, sem):
        cid = lax.axis_index("c")
        sid = lax.axis_index("s")
        wid = sid * NC + cid
        base = wid * ppw
        sp_base = (base // P) * (H * W)

        pltpu.sync_copy(grid_hbm.at[pl.ds(base * 2, 2 * ppw)], gxy)

        lane = lax.iota(jnp.int32, 16)
        fW = jnp.float32(W)
        fH = jnp.float32(H)

        def compute_vec(v, carry):
            gi = v * 32 + 2 * lane
            x = plsc.load_gather(gxy, [gi])
            y = plsc.load_gather(gxy, [gi + 1])
            fx = x * fW - 0.5
            fy = y * fH - 0.5
            txi = fx.astype(jnp.int32)
            tyi = fy.astype(jnp.int32)
            ix0 = txi - jnp.where(txi.astype(jnp.float32) > fx, 1, 0)
            iy0 = tyi - jnp.where(tyi.astype(jnp.float32) > fy, 1, 0)
            dx = fx - ix0.astype(jnp.float32)
            dy = fy - iy0.astype(jnp.float32)
            one = jnp.float32(1.0)
            sc = v // 8
            off = (v % 8) * _LANES
            for c, (oy, ox, wgt) in enumerate((
                    (0, 0, lambda: (one - dy) * (one - dx)),
                    (1, 0, lambda: dy * (one - dx)),
                    (0, 1, lambda: (one - dy) * dx),
                    (1, 1, lambda: dy * dx))):
                iy = iy0 + oy
                ix = ix0 + ox
                valid = ((iy >= 0) & (iy <= H - 1)) & ((ix >= 0) & (ix <= W - 1))
                w = jnp.where(valid, wgt(), jnp.float32(0.0))
                iyc = jnp.clip(iy, 0, H - 1)
                ixc = jnp.clip(ix, 0, W - 1)
                flat = sp_base + iyc * W + ixc
                idxb[c, sc, pl.ds(off, _LANES)] = flat
                wb[pl.ds(c * ppw + v * _LANES, _LANES)] = w
            return carry

        lax.fori_loop(0, nvec, compute_vec, 0)

        def do_sub(sch, carry):
            descs = [
                pltpu.async_copy(feat_hbm.at[idxb.at[c, sch]], rows.at[c], sem)
                for c in range(4)
            ]
            for d in descs:
                d.wait()

            wbase = sch * _SUB

            def point(i, c2):
                ws = []
                for c in range(4):
                    widx = jnp.full((16,), c * ppw + wbase + i, jnp.int32)
                    ws.append(plsc.load_gather(wb, [widx]))
                for j in range(cvec):
                    sl = pl.ds(j * _LANES, _LANES)
                    acc = ws[0] * rows[0, i, sl]
                    acc = acc + ws[1] * rows[1, i, sl]
                    acc = acc + ws[2] * rows[2, i, sl]
                    acc = acc + ws[3] * rows[3, i, sl]
                    outb[i, sl] = acc
                return c2

            lax.fori_loop(0, _SUB, point, 0)
            pltpu.sync_copy(outb, out_hbm.at[pl.ds(base + sch * _SUB, _SUB)])
            return carry

        lax.fori_loop(0, nsub, do_sub, 0)

    return point_sample


def kernel(features, grid):
    B, H, W, C = features.shape
    P = grid.shape[1]
    feat = features.reshape(B * H * W, C)
    gridf = grid.reshape(B * P * 2)
    out = _build(B, H, W, C, P)(feat, gridf)
    return out.reshape(B, P, C)
